# baseline (speedup 1.0000x reference)
import functools

import jax
import jax.numpy as jnp
from jax import lax
from jax.experimental import pallas as pl
from jax.experimental.pallas import tpu as pltpu
from jax.experimental.pallas import tpu_sc as plsc

N = 10000
D = 128
E = 320000
NC = 2
NS = 16
NW = NC * NS
CH = 120
UN = 3
NITER = 28
NSUB = UN * NITER
EPAD = NW * NSUB * CH
NPAD = 10240
ZR = NPAD // NS

_mesh = plsc.VectorSubcoreMesh(core_axis_name="c", subcore_axis_name="s")


@functools.partial(
    pl.kernel,
    out_type=jax.ShapeDtypeStruct((NC, NPAD), jnp.float32),
    mesh=_mesh,
    scratch_types=[
        pltpu.VMEM((NITER, UN, 2, CH), jnp.int32),
        pltpu.VMEM((CH,), jnp.float32),
        pltpu.VMEM((ZR,), jnp.float32),
        pltpu.VMEM_SHARED((NPAD,), jnp.float32),
    ],
)
def _sc_count(eidx_hbm, cnt_hbm, idx_v, ones_v, zed_v, acc_sh):
    c = lax.axis_index("c")
    s = lax.axis_index("s")
    w = c * NS + s
    for i in range(CH // 16):
        ones_v[pl.ds(i * 16, 16)] = jnp.ones((16,), jnp.float32)
    for i in range(ZR // 16):
        zed_v[pl.ds(i * 16, 16)] = jnp.zeros((16,), jnp.float32)
    pltpu.sync_copy(zed_v, acc_sh.at[pl.ds(s * ZR, ZR)])
    pltpu.sync_copy(eidx_hbm.at[w], idx_v)
    plsc.subcore_barrier()

    def body(t, carry):
        for u in range(UN):
            pltpu.sync_copy(ones_v, acc_sh.at[idx_v.at[t, u, 1]], add=True)
        return carry

    lax.fori_loop(0, NITER, body, 0)
    plsc.subcore_barrier()
    pltpu.sync_copy(acc_sh.at[pl.ds(s * ZR, ZR)], cnt_hbm.at[c, pl.ds(s * ZR, ZR)])


@functools.partial(
    pl.kernel,
    out_type=jax.ShapeDtypeStruct((NC, NPAD, D), jnp.float32),
    mesh=_mesh,
    scratch_types=[
        pltpu.VMEM((2, UN, 2, CH), jnp.int32),
        pltpu.VMEM((CH, D), jnp.float32),
        pltpu.VMEM((CH, D), jnp.float32),
        pltpu.VMEM((CH, D), jnp.float32),
        pltpu.VMEM_SHARED((NPAD, D), jnp.float32),
        pltpu.SemaphoreType.DMA,
        pltpu.SemaphoreType.DMA,
        pltpu.SemaphoreType.DMA,
        pltpu.SemaphoreType.DMA,
        pltpu.SemaphoreType.DMA,
        pltpu.SemaphoreType.DMA,
        pltpu.SemaphoreType.DMA,
    ],
)
def _sc_scatter(g_hbm, eidx_hbm, zrow_hbm, out_hbm,
                ring_v, r0, r1, r2, acc_sh,
                sg0, sg1, sg2, ss0, ss1, ss2, si):
    c = lax.axis_index("c")
    s = lax.axis_index("s")
    w = c * NS + s
    tile_idx = eidx_hbm.at[w]
    pltpu.sync_copy(zrow_hbm, acc_sh.at[pl.ds(s * ZR, ZR)])
    pltpu.sync_copy(tile_idx.at[0], ring_v.at[0])
    plsc.subcore_barrier()

    pltpu.async_copy(g_hbm.at[ring_v.at[0, 0, 0]], r0, sg0)
    pltpu.async_copy(g_hbm.at[ring_v.at[0, 1, 0]], r1, sg1)

    def it(t, carry):
        q = t % 2
        qn = (t + 1) % 2
        live = t + 1 < NITER

        pltpu.make_async_copy(g_hbm.at[ring_v.at[q, 0, 0]], r0, sg0).wait()
        pltpu.async_copy(r0, acc_sh.at[ring_v.at[q, 0, 1]], ss0, add=True)

        @pl.when(t > 0)
        def _():
            pltpu.make_async_copy(r2, acc_sh.at[ring_v.at[qn, 2, 1]], ss2).wait()

        @pl.when(live)
        def _():
            pltpu.async_copy(tile_idx.at[t + 1], ring_v.at[qn], si)
        pltpu.async_copy(g_hbm.at[ring_v.at[q, 2, 0]], r2, sg2)

        pltpu.make_async_copy(g_hbm.at[ring_v.at[q, 1, 0]], r1, sg1).wait()
        pltpu.async_copy(r1, acc_sh.at[ring_v.at[q, 1, 1]], ss1, add=True)
        pltpu.make_async_copy(r0, acc_sh.at[ring_v.at[q, 0, 1]], ss0).wait()

        @pl.when(live)
        def _():
            pltpu.make_async_copy(tile_idx.at[t + 1], ring_v.at[qn], si).wait()
            pltpu.async_copy(g_hbm.at[ring_v.at[qn, 0, 0]], r0, sg0)

        pltpu.make_async_copy(g_hbm.at[ring_v.at[q, 2, 0]], r2, sg2).wait()
        pltpu.async_copy(r2, acc_sh.at[ring_v.at[q, 2, 1]], ss2, add=True)
        pltpu.make_async_copy(r1, acc_sh.at[ring_v.at[q, 1, 1]], ss1).wait()

        @pl.when(live)
        def _():
            pltpu.async_copy(g_hbm.at[ring_v.at[qn, 1, 0]], r1, sg1)

        return carry

    lax.fori_loop(0, NITER, it, 0)
    pltpu.make_async_copy(r2, acc_sh.at[ring_v.at[(NITER - 1) % 2, 2, 1]],
                          ss2).wait()
    plsc.subcore_barrier()
    pltpu.sync_copy(acc_sh.at[pl.ds(s * ZR, ZR)], out_hbm.at[c, pl.ds(s * ZR, ZR)])


RB = 5000


def _dis(cnt_ref):
    return lax.rsqrt(cnt_ref[0] + cnt_ref[1] + 1.0)


def _pre_body(x_ref, w_ref, cnt_ref, g_ref):
    g_ref[...] = jnp.dot(x_ref[...], w_ref[...],
                         preferred_element_type=jnp.float32) * _dis(cnt_ref)


def _mid_body(s_ref, g_ref, cnt_ref, w_ref, b_ref, out_ref):
    dis = _dis(cnt_ref)
    p = dis * (s_ref[0] + s_ref[1] + g_ref[...]) + b_ref[...]
    h = jnp.maximum(p, 0.0)
    out_ref[...] = jnp.dot(h, w_ref[...],
                           preferred_element_type=jnp.float32) * dis


def _post_body(s_ref, g_ref, cnt_ref, b_ref, out_ref):
    dis = _dis(cnt_ref)
    out_ref[...] = dis * (s_ref[0] + s_ref[1] + g_ref[...]) + b_ref[...]


_s_spec = pl.BlockSpec((NC, RB, D), lambda r: (0, r, 0))
_row_spec = pl.BlockSpec((RB, D), lambda r: (r, 0))
_w_spec = pl.BlockSpec((D, D), lambda r: (0, 0))
_cnt_spec = pl.BlockSpec((NC, RB, 1), lambda r: (0, r, 0))
_b_spec = pl.BlockSpec((1, D), lambda r: (0, 0))
_out_row = jax.ShapeDtypeStruct((N, D), jnp.float32)

_pre = pl.pallas_call(
    _pre_body,
    grid=(N // RB,),
    in_specs=[_row_spec, _w_spec, _cnt_spec],
    out_specs=_row_spec,
    out_shape=_out_row,
)

_mid = pl.pallas_call(
    _mid_body,
    grid=(N // RB,),
    in_specs=[_s_spec, _row_spec, _cnt_spec, _w_spec, _b_spec],
    out_specs=_row_spec,
    out_shape=_out_row,
)

_post = pl.pallas_call(
    _post_body,
    grid=(N // RB,),
    in_specs=[_s_spec, _row_spec, _cnt_spec, _b_spec],
    out_specs=_row_spec,
    out_shape=_out_row,
)


@jax.jit
def kernel(x, edge_index, W1, b1, W2, b2):
    ei = edge_index.astype(jnp.int32)
    npd = EPAD - E
    pad_src = jnp.arange(npd, dtype=jnp.int32) % N
    pad_dst = N + (jnp.arange(npd, dtype=jnp.int32) % (NPAD - N))
    srcp = jnp.concatenate([ei[0], pad_src]).reshape(NW, NITER, UN, CH)
    dstp = jnp.concatenate([ei[1], pad_dst]).reshape(NW, NITER, UN, CH)
    eidx = jnp.stack([srcp, dstp], axis=3)
    zrow = jnp.zeros((ZR, D), jnp.float32)
    b1r = b1.reshape(1, D)
    b2r = b2.reshape(1, D)

    cnt = _sc_count(eidx)
    cnt3 = cnt.reshape(NC, NPAD, 1)
    g1 = _pre(x, W1, cnt3)
    s1 = _sc_scatter(g1, eidx, zrow)
    g2 = _mid(s1, g1, cnt3, W2, b1r)
    s2 = _sc_scatter(g2, eidx, zrow)
    return _post(s2, g2, cnt3, b2r)


# speedup vs baseline: 1.0034x; 1.0034x over previous
import functools

import jax
import jax.numpy as jnp
from jax import lax
from jax.experimental import pallas as pl
from jax.experimental.pallas import tpu as pltpu
from jax.experimental.pallas import tpu_sc as plsc

N = 10000
D = 128
E = 320000
NC = 2
NS = 16
NW = NC * NS
CH = 112
UN = 3
NITER = 30
NSUB = UN * NITER
EPAD = NW * NSUB * CH
NPAD = 10240
ZR = NPAD // NS

_mesh = plsc.VectorSubcoreMesh(core_axis_name="c", subcore_axis_name="s")


@functools.partial(
    pl.kernel,
    out_type=jax.ShapeDtypeStruct((NC, NPAD), jnp.float32),
    mesh=_mesh,
    scratch_types=[
        pltpu.VMEM((NITER, UN, 2, CH), jnp.int32),
        pltpu.VMEM((CH,), jnp.float32),
        pltpu.VMEM((ZR,), jnp.float32),
        pltpu.VMEM_SHARED((NPAD,), jnp.float32),
    ],
)
def _sc_count(eidx_hbm, cnt_hbm, idx_v, ones_v, zed_v, acc_sh):
    c = lax.axis_index("c")
    s = lax.axis_index("s")
    w = c * NS + s
    for i in range(CH // 16):
        ones_v[pl.ds(i * 16, 16)] = jnp.ones((16,), jnp.float32)
    for i in range(ZR // 16):
        zed_v[pl.ds(i * 16, 16)] = jnp.zeros((16,), jnp.float32)
    pltpu.sync_copy(zed_v, acc_sh.at[pl.ds(s * ZR, ZR)])
    pltpu.sync_copy(eidx_hbm.at[w], idx_v)
    plsc.subcore_barrier()

    def body(t, carry):
        for u in range(UN):
            pltpu.sync_copy(ones_v, acc_sh.at[idx_v.at[t, u, 1]], add=True)
        return carry

    lax.fori_loop(0, NITER, body, 0)
    plsc.subcore_barrier()
    pltpu.sync_copy(acc_sh.at[pl.ds(s * ZR, ZR)], cnt_hbm.at[c, pl.ds(s * ZR, ZR)])


@functools.partial(
    pl.kernel,
    out_type=jax.ShapeDtypeStruct((NC, NPAD, D), jnp.float32),
    mesh=_mesh,
    scratch_types=[
        pltpu.VMEM((2, UN, 2, CH), jnp.int32),
        pltpu.VMEM((CH, D), jnp.float32),
        pltpu.VMEM((CH, D), jnp.float32),
        pltpu.VMEM((CH, D), jnp.float32),
        pltpu.VMEM_SHARED((NPAD, D), jnp.float32),
        pltpu.SemaphoreType.DMA,
        pltpu.SemaphoreType.DMA,
        pltpu.SemaphoreType.DMA,
        pltpu.SemaphoreType.DMA,
        pltpu.SemaphoreType.DMA,
        pltpu.SemaphoreType.DMA,
        pltpu.SemaphoreType.DMA,
    ],
)
def _sc_scatter(g_hbm, eidx_hbm, zrow_hbm, out_hbm,
                ring_v, r0, r1, r2, acc_sh,
                sg0, sg1, sg2, ss0, ss1, ss2, si):
    c = lax.axis_index("c")
    s = lax.axis_index("s")
    w = c * NS + s
    tile_idx = eidx_hbm.at[w]
    pltpu.sync_copy(zrow_hbm, acc_sh.at[pl.ds(s * ZR, ZR)])
    pltpu.sync_copy(tile_idx.at[0], ring_v.at[0])
    plsc.subcore_barrier()

    pltpu.async_copy(g_hbm.at[ring_v.at[0, 0, 0]], r0, sg0)
    pltpu.async_copy(g_hbm.at[ring_v.at[0, 1, 0]], r1, sg1)

    def it(t, carry):
        q = t % 2
        qn = (t + 1) % 2
        live = t + 1 < NITER

        pltpu.make_async_copy(g_hbm.at[ring_v.at[q, 0, 0]], r0, sg0).wait()
        pltpu.async_copy(r0, acc_sh.at[ring_v.at[q, 0, 1]], ss0, add=True)

        @pl.when(t > 0)
        def _():
            pltpu.make_async_copy(r2, acc_sh.at[ring_v.at[qn, 2, 1]], ss2).wait()

        @pl.when(live)
        def _():
            pltpu.async_copy(tile_idx.at[t + 1], ring_v.at[qn], si)
        pltpu.async_copy(g_hbm.at[ring_v.at[q, 2, 0]], r2, sg2)

        pltpu.make_async_copy(g_hbm.at[ring_v.at[q, 1, 0]], r1, sg1).wait()
        pltpu.async_copy(r1, acc_sh.at[ring_v.at[q, 1, 1]], ss1, add=True)
        pltpu.make_async_copy(r0, acc_sh.at[ring_v.at[q, 0, 1]], ss0).wait()

        @pl.when(live)
        def _():
            pltpu.make_async_copy(tile_idx.at[t + 1], ring_v.at[qn], si).wait()
            pltpu.async_copy(g_hbm.at[ring_v.at[qn, 0, 0]], r0, sg0)

        pltpu.make_async_copy(g_hbm.at[ring_v.at[q, 2, 0]], r2, sg2).wait()
        pltpu.async_copy(r2, acc_sh.at[ring_v.at[q, 2, 1]], ss2, add=True)
        pltpu.make_async_copy(r1, acc_sh.at[ring_v.at[q, 1, 1]], ss1).wait()

        @pl.when(live)
        def _():
            pltpu.async_copy(g_hbm.at[ring_v.at[qn, 1, 0]], r1, sg1)

        return carry

    lax.fori_loop(0, NITER, it, 0)
    pltpu.make_async_copy(r2, acc_sh.at[ring_v.at[(NITER - 1) % 2, 2, 1]],
                          ss2).wait()
    plsc.subcore_barrier()
    pltpu.sync_copy(acc_sh.at[pl.ds(s * ZR, ZR)], out_hbm.at[c, pl.ds(s * ZR, ZR)])


RB = 5000


def _dis(cnt_ref):
    return lax.rsqrt(cnt_ref[0] + cnt_ref[1] + 1.0)


def _pre_body(x_ref, w_ref, cnt_ref, g_ref):
    g_ref[...] = jnp.dot(x_ref[...], w_ref[...],
                         preferred_element_type=jnp.float32) * _dis(cnt_ref)


def _mid_body(s_ref, g_ref, cnt_ref, w_ref, b_ref, out_ref):
    dis = _dis(cnt_ref)
    p = dis * (s_ref[0] + s_ref[1] + g_ref[...]) + b_ref[...]
    h = jnp.maximum(p, 0.0)
    out_ref[...] = jnp.dot(h, w_ref[...],
                           preferred_element_type=jnp.float32) * dis


def _post_body(s_ref, g_ref, cnt_ref, b_ref, out_ref):
    dis = _dis(cnt_ref)
    out_ref[...] = dis * (s_ref[0] + s_ref[1] + g_ref[...]) + b_ref[...]


_s_spec = pl.BlockSpec((NC, RB, D), lambda r: (0, r, 0))
_row_spec = pl.BlockSpec((RB, D), lambda r: (r, 0))
_w_spec = pl.BlockSpec((D, D), lambda r: (0, 0))
_cnt_spec = pl.BlockSpec((NC, RB, 1), lambda r: (0, r, 0))
_b_spec = pl.BlockSpec((1, D), lambda r: (0, 0))
_out_row = jax.ShapeDtypeStruct((N, D), jnp.float32)

_pre = pl.pallas_call(
    _pre_body,
    grid=(N // RB,),
    in_specs=[_row_spec, _w_spec, _cnt_spec],
    out_specs=_row_spec,
    out_shape=_out_row,
)

_mid = pl.pallas_call(
    _mid_body,
    grid=(N // RB,),
    in_specs=[_s_spec, _row_spec, _cnt_spec, _w_spec, _b_spec],
    out_specs=_row_spec,
    out_shape=_out_row,
)

_post = pl.pallas_call(
    _post_body,
    grid=(N // RB,),
    in_specs=[_s_spec, _row_spec, _cnt_spec, _b_spec],
    out_specs=_row_spec,
    out_shape=_out_row,
)


@jax.jit
def kernel(x, edge_index, W1, b1, W2, b2):
    ei = edge_index.astype(jnp.int32)
    npd = EPAD - E
    pad_src = jnp.arange(npd, dtype=jnp.int32) % N
    pad_dst = N + (jnp.arange(npd, dtype=jnp.int32) % (NPAD - N))
    srcp = jnp.concatenate([ei[0], pad_src]).reshape(NW, NITER, UN, CH)
    dstp = jnp.concatenate([ei[1], pad_dst]).reshape(NW, NITER, UN, CH)
    eidx = jnp.stack([srcp, dstp], axis=3)
    zrow = jnp.zeros((ZR, D), jnp.float32)
    b1r = b1.reshape(1, D)
    b2r = b2.reshape(1, D)

    cnt = _sc_count(eidx)
    cnt3 = cnt.reshape(NC, NPAD, 1)
    g1 = _pre(x, W1, cnt3)
    s1 = _sc_scatter(g1, eidx, zrow)
    g2 = _mid(s1, g1, cnt3, W2, b1r)
    s2 = _sc_scatter(g2, eidx, zrow)
    return _post(s2, g2, cnt3, b2r)
